# Initial kernel scaffold; baseline (speedup 1.0000x reference)
import jax
import jax.numpy as jnp
from jax.experimental import pallas as pl


def kernel(Z, indices, weights, num_nodes):
    raise NotImplementedError("write your pallas kernel here")


# speedup vs baseline: 1.1086x; 1.1086x over previous
import functools

import jax
import jax.numpy as jnp
from jax import lax
from jax.experimental import pallas as pl
from jax.experimental.pallas import tpu as pltpu
from jax.experimental.pallas import tpu_sc as plsc

LAM = 0.001

_mesh = plsc.VectorSubcoreMesh(core_axis_name="c", subcore_axis_name="s")

N = 10000
K = 32
D = 128
D2 = D // 2
NPAD = 10240
G16 = NPAD // 16
RPT = NPAD // 32
EB = (G16 * K) // 128
NCH = RPT // 64


def _newton_rsqrt(x):
    xi = lax.bitcast_convert_type(x, jnp.int32)
    yi = jnp.int32(0x5F3759DF) - lax.shift_right_logical(xi, jnp.int32(1))
    y = lax.bitcast_convert_type(yi, jnp.float32)
    for _ in range(3):
        y = y * (1.5 - 0.5 * x * y * y)
    return y


EBT = (NPAD * K) // (32 * 128)


@functools.partial(
    pl.kernel,
    out_type=jax.ShapeDtypeStruct((2, G16, 16), jnp.float32),
    mesh=_mesh,
    compiler_params=pltpu.CompilerParams(use_tc_tiling_on_sc=False),
    scratch_types=[
        pltpu.VMEM((16, 128, 16), jnp.float32),
        pltpu.VMEM((EBT, 128), jnp.int32),
        pltpu.VMEM_SHARED((G16, 16), jnp.float32),
    ],
)
def _phase_a1(w16c_hbm, idxc_hbm, w16r_hbm, idxr_hbm, zeros_hbm, deg_hbm,
              pay_v, idx_v, deg_sh):
    c = lax.axis_index("c")
    s = lax.axis_index("s")
    wid = s * 2 + c

    pltpu.sync_copy(zeros_hbm.at[pl.ds(s * 40, 40)],
                    deg_sh.at[pl.ds(s * 40, 40)])
    plsc.subcore_barrier()

    for w16_hbm, id_hbm in ((w16c_hbm, idxc_hbm), (w16r_hbm, idxr_hbm)):
        pltpu.sync_copy(id_hbm.at[wid], idx_v)

        def chunk_body(ch, carry):
            pltpu.sync_copy(w16_hbm.at[wid, pl.ds(ch * 16, 16)], pay_v)
            for j in range(16):
                pltpu.sync_copy(pay_v.at[j],
                                deg_sh.at[idx_v.at[ch * 16 + j]], add=True)
            return carry

        lax.fori_loop(0, EBT // 16, chunk_body, jnp.int32(0))
    plsc.subcore_barrier()

    pltpu.sync_copy(deg_sh.at[pl.ds(s * 40, 40)],
                    deg_hbm.at[c, pl.ds(s * 40, 40)])


@functools.partial(
    pl.kernel,
    out_type=(
        jax.ShapeDtypeStruct((NPAD, D2), jnp.float32),
        jax.ShapeDtypeStruct((NPAD, D2), jnp.float32),
        jax.ShapeDtypeStruct((NPAD,), jnp.float32),
    ),
    mesh=_mesh,
    compiler_params=pltpu.CompilerParams(use_tc_tiling_on_sc=False),
    scratch_types=[
        pltpu.VMEM((RPT, D2), jnp.float32),
        pltpu.VMEM((RPT, D2), jnp.float32),
        pltpu.VMEM((RPT,), jnp.float32),
        pltpu.VMEM((RPT,), jnp.float32),
        pltpu.VMEM((RPT,), jnp.float32),
    ],
)
def _phase_a2(z0_hbm, z1_hbm, degp_hbm, zn0_hbm, zn1_hbm, deg_hbm,
              z0_v, z1_v, dp_v, dq_v, dinv_v):
    c = lax.axis_index("c")
    s = lax.axis_index("s")
    wid = s * 2 + c

    pltpu.sync_copy(degp_hbm.at[0, pl.ds(wid * RPT, RPT)], dp_v)
    pltpu.sync_copy(degp_hbm.at[1, pl.ds(wid * RPT, RPT)], dq_v)
    pltpu.sync_copy(z0_hbm.at[pl.ds(wid * RPT, RPT)], z0_v)
    pltpu.sync_copy(z1_hbm.at[pl.ds(wid * RPT, RPT)], z1_v)
    for j in range(RPT // 16):
        sl = pl.ds(j * 16, 16)
        dh = (dp_v[sl] + dq_v[sl]) * 0.5
        dp_v[sl] = dh
        dinv_v[sl] = _newton_rsqrt(dh + 1e-8)

    def grp_body(g, carry):
        dvec = dinv_v[pl.ds(pl.multiple_of(g * 16, 16), 16)]
        for l in range(16):
            dv = dvec[l]
            r = g * 16 + l
            for d in range(D2 // 16):
                sl = pl.ds(d * 16, 16)
                z0_v[r, sl] = dv * z0_v[r, sl]
                z1_v[r, sl] = dv * z1_v[r, sl]
        return carry

    lax.fori_loop(0, RPT // 16, grp_body, jnp.int32(0))
    pltpu.sync_copy(z0_v, zn0_hbm.at[pl.ds(wid * RPT, RPT)])
    pltpu.sync_copy(z1_v, zn1_hbm.at[pl.ds(wid * RPT, RPT)])
    pltpu.sync_copy(dp_v, deg_hbm.at[pl.ds(wid * RPT, RPT)])


@functools.partial(
    pl.kernel,
    out_type=jax.ShapeDtypeStruct((2, 2, NPAD, D2), jnp.float32),
    mesh=_mesh,
    compiler_params=pltpu.CompilerParams(use_tc_tiling_on_sc=False),
    scratch_types=[
        pltpu.VMEM((64, K), jnp.int32),
        pltpu.VMEM((64, K), jnp.float32),
        pltpu.VMEM((64, D2), jnp.float32),
        pltpu.VMEM((64, D2), jnp.float32),
        pltpu.VMEM((K, D2), jnp.float32),
        pltpu.VMEM((K, D2), jnp.float32),
        pltpu.VMEM((NCH, 64), jnp.int32),
        pltpu.VMEM_SHARED((NPAD, D2), jnp.float32),
        pltpu.SemaphoreType.DMA,
    ],
)
def _phase_b(zn0_hbm, zn1_hbm, ind_hbm, w_hbm, selfidx_hbm, az_hbm,
             ind_v, w_v, zn_v, azo_v, gbuf, sbuf, sidx_v, az_sh, sem):
    c = lax.axis_index("c")
    s = lax.axis_index("s")
    wid = s * 2 + c

    pltpu.sync_copy(selfidx_hbm.at[wid], sidx_v)

    for h, zn_hbm in enumerate((zn0_hbm, zn1_hbm)):
        for k in range(K):
            for d in range(D2 // 16):
                sbuf[k, pl.ds(d * 16, 16)] = jnp.zeros((16,), jnp.float32)
        for j in range(G16 // K):
            pltpu.sync_copy(sbuf, az_sh.at[pl.ds(s * G16 + j * K, K)])
        plsc.subcore_barrier()

        def chunk_body(ch, carry):
            base = wid * RPT + ch * 64
            pltpu.sync_copy(ind_hbm.at[pl.ds(base, 64)], ind_v)
            pltpu.sync_copy(w_hbm.at[pl.ds(base, 64)], w_v)
            pltpu.sync_copy(zn_hbm.at[pl.ds(base, 64)], zn_v)

            def row_body(r, carry2):
                pltpu.async_copy(zn_hbm.at[ind_v.at[r]], gbuf, sem).wait()
                acc = [jnp.zeros((16,), jnp.float32) for _ in range(D2 // 16)]
                zrow = [zn_v[r, pl.ds(d * 16, 16)] for d in range(D2 // 16)]
                wrow = [w_v[r, pl.ds(0, 16)], w_v[r, pl.ds(16, 16)]]
                for k in range(K):
                    wk = wrow[k // 16][k % 16] * 0.5
                    for d in range(D2 // 16):
                        sl = pl.ds(d * 16, 16)
                        acc[d] = acc[d] + wk * gbuf[k, sl]
                        sbuf[k, sl] = wk * zrow[d]
                for d in range(D2 // 16):
                    azo_v[r, pl.ds(d * 16, 16)] = acc[d]
                pltpu.sync_copy(sbuf, az_sh.at[ind_v.at[r]], add=True)
                return carry2

            lax.fori_loop(0, 64, row_body, jnp.int32(0))
            pltpu.sync_copy(azo_v, az_sh.at[sidx_v.at[ch]], add=True)
            return carry

        lax.fori_loop(0, NCH, chunk_body, jnp.int32(0))
        plsc.subcore_barrier()

        pltpu.sync_copy(az_sh.at[pl.ds(s * G16, G16)],
                        az_hbm.at[h, c, pl.ds(s * G16, G16)])


@functools.partial(
    pl.kernel,
    out_type=jax.ShapeDtypeStruct((32, 16), jnp.float32),
    mesh=_mesh,
    compiler_params=pltpu.CompilerParams(use_tc_tiling_on_sc=False),
    scratch_types=[
        pltpu.VMEM((RPT, D2), jnp.float32),
        pltpu.VMEM((RPT, D2), jnp.float32),
        pltpu.VMEM((RPT, D2), jnp.float32),
        pltpu.VMEM((RPT,), jnp.float32),
        pltpu.VMEM((1, 16), jnp.float32),
    ],
)
def _phase_c(zn0_hbm, zn1_hbm, deg_hbm, az_hbm, out_hbm,
             zn_v, a0_v, a1_v, deg_v, st_v):
    c = lax.axis_index("c")
    s = lax.axis_index("s")
    wid = s * 2 + c

    pltpu.sync_copy(deg_hbm.at[pl.ds(wid * RPT, RPT)], deg_v)
    acc_total = jnp.zeros((16,), jnp.float32)
    for h, zn_hbm in enumerate((zn0_hbm, zn1_hbm)):
        pltpu.sync_copy(zn_hbm.at[pl.ds(wid * RPT, RPT)], zn_v)
        pltpu.sync_copy(az_hbm.at[h, 0, pl.ds(wid * RPT, RPT)], a0_v)
        pltpu.sync_copy(az_hbm.at[h, 1, pl.ds(wid * RPT, RPT)], a1_v)

        def grp_body(g, acc):
            dvec = deg_v[pl.ds(pl.multiple_of(g * 16, 16), 16)]
            for l in range(16):
                dr = dvec[l]
                r = g * 16 + l
                for d in range(D2 // 16):
                    sl = pl.ds(d * 16, 16)
                    t = dr * zn_v[r, sl] - a0_v[r, sl] - a1_v[r, sl]
                    acc = acc + t * t
            return acc

        acc_total = lax.fori_loop(0, RPT // 16, grp_body, acc_total)
    st_v[0, :] = acc_total
    pltpu.sync_copy(st_v.at[0], out_hbm.at[wid])


def kernel(Z, indices, weights, num_nodes):
    pad = NPAD - N
    Zp = jnp.pad(Z, ((0, pad), (0, 0)))
    indp = jnp.pad(indices.astype(jnp.int32), ((0, pad), (0, 0)))
    wp = jnp.pad(weights, ((0, pad), (0, 0)))

    wf = wp.reshape(-1)
    cf = indp.reshape(-1)
    rf = jnp.repeat(jnp.arange(NPAD, dtype=jnp.int32), K)
    eye16 = jnp.eye(16, dtype=jnp.float32)
    w16c = (wf[:, None] * eye16[cf % 16]).reshape(32, EBT, 128, 16)
    idxc = (cf // 16).reshape(32, EBT, 128)
    w16r = (wf[:, None] * eye16[rf % 16]).reshape(32, EBT, 128, 16)
    idxr = (rf // 16).reshape(32, EBT, 128)
    selfidx = jnp.arange(NPAD, dtype=jnp.int32).reshape(32, NCH, 64)
    zeros1 = jnp.zeros((G16, 16), jnp.float32)

    degp = _phase_a1(w16c, idxc, w16r, idxr, zeros1)
    zn0, zn1, deg = _phase_a2(Zp[:, :D2], Zp[:, D2:], degp.reshape(2, NPAD))
    az = _phase_b(zn0, zn1, indp, wp, selfidx)
    partials = _phase_c(zn0, zn1, deg, az)
    smooth = jnp.sum(partials)
    return LAM * smooth / (num_nodes + 1e-08)


# speedup vs baseline: 2.8314x; 2.5541x over previous
import functools

import jax
import jax.numpy as jnp
from jax import lax
from jax.experimental import pallas as pl
from jax.experimental.pallas import tpu as pltpu
from jax.experimental.pallas import tpu_sc as plsc

LAM = 0.001

_mesh = plsc.VectorSubcoreMesh(core_axis_name="c", subcore_axis_name="s")

N = 10000
K = 32
D = 128
D2 = D // 2
NPAD = 10240
G16 = NPAD // 16
RPT = NPAD // 32
EB = (G16 * K) // 128
NCH = RPT // 64


def _newton_rsqrt(x):
    xi = lax.bitcast_convert_type(x, jnp.int32)
    yi = jnp.int32(0x5F3759DF) - lax.shift_right_logical(xi, jnp.int32(1))
    y = lax.bitcast_convert_type(yi, jnp.float32)
    for _ in range(3):
        y = y * (1.5 - 0.5 * x * y * y)
    return y


EBT = (NPAD * K) // (32 * 128)


@functools.partial(
    pl.kernel,
    out_type=jax.ShapeDtypeStruct((2, G16, 16), jnp.float32),
    mesh=_mesh,
    compiler_params=pltpu.CompilerParams(use_tc_tiling_on_sc=False),
    scratch_types=[
        pltpu.VMEM((16, 128, 16), jnp.float32),
        pltpu.VMEM((EBT, 128), jnp.int32),
        pltpu.VMEM_SHARED((G16, 16), jnp.float32),
    ],
)
def _phase_a1(w16c_hbm, idxc_hbm, w16r_hbm, idxr_hbm, zeros_hbm, deg_hbm,
              pay_v, idx_v, deg_sh):
    c = lax.axis_index("c")
    s = lax.axis_index("s")
    wid = s * 2 + c

    pltpu.sync_copy(zeros_hbm.at[pl.ds(s * 40, 40)],
                    deg_sh.at[pl.ds(s * 40, 40)])
    plsc.subcore_barrier()

    for w16_hbm, id_hbm in ((w16c_hbm, idxc_hbm), (w16r_hbm, idxr_hbm)):
        pltpu.sync_copy(id_hbm.at[wid], idx_v)

        def chunk_body(ch, carry):
            pltpu.sync_copy(w16_hbm.at[wid, pl.ds(ch * 16, 16)], pay_v)
            for j in range(16):
                pltpu.sync_copy(pay_v.at[j],
                                deg_sh.at[idx_v.at[ch * 16 + j]], add=True)
            return carry

        lax.fori_loop(0, EBT // 16, chunk_body, jnp.int32(0))
    plsc.subcore_barrier()

    pltpu.sync_copy(deg_sh.at[pl.ds(s * 40, 40)],
                    deg_hbm.at[c, pl.ds(s * 40, 40)])


@functools.partial(
    pl.kernel,
    out_type=(
        jax.ShapeDtypeStruct((NPAD, D2), jnp.float32),
        jax.ShapeDtypeStruct((NPAD, D2), jnp.float32),
        jax.ShapeDtypeStruct((NPAD,), jnp.float32),
    ),
    mesh=_mesh,
    compiler_params=pltpu.CompilerParams(use_tc_tiling_on_sc=False),
    scratch_types=[
        pltpu.VMEM((RPT, D2), jnp.float32),
        pltpu.VMEM((RPT, D2), jnp.float32),
        pltpu.VMEM((RPT,), jnp.float32),
        pltpu.VMEM((RPT,), jnp.float32),
        pltpu.VMEM((RPT,), jnp.float32),
    ],
)
def _phase_a2(z0_hbm, z1_hbm, degp_hbm, zn0_hbm, zn1_hbm, deg_hbm,
              z0_v, z1_v, dp_v, dq_v, dinv_v):
    c = lax.axis_index("c")
    s = lax.axis_index("s")
    wid = s * 2 + c

    pltpu.sync_copy(degp_hbm.at[0, pl.ds(wid * RPT, RPT)], dp_v)
    pltpu.sync_copy(degp_hbm.at[1, pl.ds(wid * RPT, RPT)], dq_v)
    pltpu.sync_copy(z0_hbm.at[pl.ds(wid * RPT, RPT)], z0_v)
    pltpu.sync_copy(z1_hbm.at[pl.ds(wid * RPT, RPT)], z1_v)
    for j in range(RPT // 16):
        sl = pl.ds(j * 16, 16)
        dh = (dp_v[sl] + dq_v[sl]) * 0.5
        dp_v[sl] = dh
        dinv_v[sl] = _newton_rsqrt(dh + 1e-8)

    def grp_body(g, carry):
        dvec = dinv_v[pl.ds(pl.multiple_of(g * 16, 16), 16)]
        for l in range(16):
            dv = dvec[l]
            r = g * 16 + l
            for d in range(D2 // 16):
                sl = pl.ds(d * 16, 16)
                z0_v[r, sl] = dv * z0_v[r, sl]
                z1_v[r, sl] = dv * z1_v[r, sl]
        return carry

    lax.fori_loop(0, RPT // 16, grp_body, jnp.int32(0))
    pltpu.sync_copy(z0_v, zn0_hbm.at[pl.ds(wid * RPT, RPT)])
    pltpu.sync_copy(z1_v, zn1_hbm.at[pl.ds(wid * RPT, RPT)])
    pltpu.sync_copy(dp_v, deg_hbm.at[pl.ds(wid * RPT, RPT)])


@functools.partial(
    pl.kernel,
    out_type=jax.ShapeDtypeStruct((2, 2, NPAD, D2), jnp.float32),
    mesh=_mesh,
    compiler_params=pltpu.CompilerParams(use_tc_tiling_on_sc=False),
    scratch_types=[
        pltpu.VMEM((64, K), jnp.int32),
        pltpu.VMEM((64, K), jnp.float32),
        pltpu.VMEM((64, D2), jnp.float32),
        pltpu.VMEM((64, D2), jnp.float32),
        pltpu.VMEM((K, D2), jnp.float32),
        pltpu.VMEM((K, D2), jnp.float32),
        pltpu.VMEM((NCH, 64), jnp.int32),
        pltpu.VMEM_SHARED((NPAD, D2), jnp.float32),
        pltpu.SemaphoreType.DMA,
    ],
)
def _phase_b(zn0_hbm, zn1_hbm, ind_hbm, w_hbm, selfidx_hbm, az_hbm,
             ind_v, w_v, zn_v, azo_v, gbuf, sbuf, sidx_v, az_sh, sem):
    c = lax.axis_index("c")
    s = lax.axis_index("s")
    wid = s * 2 + c

    pltpu.sync_copy(selfidx_hbm.at[wid], sidx_v)

    for h, zn_hbm in enumerate((zn0_hbm, zn1_hbm)):
        for k in range(K):
            for d in range(D2 // 16):
                sbuf[k, pl.ds(d * 16, 16)] = jnp.zeros((16,), jnp.float32)
        for j in range(G16 // K):
            pltpu.sync_copy(sbuf, az_sh.at[pl.ds(s * G16 + j * K, K)])
        plsc.subcore_barrier()

        def chunk_body(ch, carry):
            base = wid * RPT + ch * 64
            pltpu.sync_copy(ind_hbm.at[pl.ds(base, 64)], ind_v)
            pltpu.sync_copy(w_hbm.at[pl.ds(base, 64)], w_v)
            pltpu.sync_copy(zn_hbm.at[pl.ds(base, 64)], zn_v)

            def row_body(r, carry2):
                pltpu.async_copy(zn_hbm.at[ind_v.at[r]], gbuf, sem).wait()
                acc = [jnp.zeros((16,), jnp.float32) for _ in range(D2 // 16)]
                zrow = [zn_v[r, pl.ds(d * 16, 16)] for d in range(D2 // 16)]
                wrow = [w_v[r, pl.ds(0, 16)], w_v[r, pl.ds(16, 16)]]
                for k in range(K):
                    wk = wrow[k // 16][k % 16] * 0.5
                    for d in range(D2 // 16):
                        sl = pl.ds(d * 16, 16)
                        acc[d] = acc[d] + wk * gbuf[k, sl]
                        sbuf[k, sl] = wk * zrow[d]
                for d in range(D2 // 16):
                    azo_v[r, pl.ds(d * 16, 16)] = acc[d]
                pltpu.sync_copy(sbuf, az_sh.at[ind_v.at[r]], add=True)
                return carry2

            lax.fori_loop(0, 64, row_body, jnp.int32(0))
            pltpu.sync_copy(azo_v, az_sh.at[sidx_v.at[ch]], add=True)
            return carry

        lax.fori_loop(0, NCH, chunk_body, jnp.int32(0))
        plsc.subcore_barrier()

        pltpu.sync_copy(az_sh.at[pl.ds(s * G16, G16)],
                        az_hbm.at[h, c, pl.ds(s * G16, G16)])


@functools.partial(
    pl.kernel,
    out_type=jax.ShapeDtypeStruct((32, 16), jnp.float32),
    mesh=_mesh,
    compiler_params=pltpu.CompilerParams(use_tc_tiling_on_sc=False),
    scratch_types=[
        pltpu.VMEM((RPT, D2), jnp.float32),
        pltpu.VMEM((RPT, D2), jnp.float32),
        pltpu.VMEM((RPT, D2), jnp.float32),
        pltpu.VMEM((RPT,), jnp.float32),
        pltpu.VMEM((1, 16), jnp.float32),
    ],
)
def _phase_c(zn0_hbm, zn1_hbm, deg_hbm, az_hbm, out_hbm,
             zn_v, a0_v, a1_v, deg_v, st_v):
    c = lax.axis_index("c")
    s = lax.axis_index("s")
    wid = s * 2 + c

    pltpu.sync_copy(deg_hbm.at[pl.ds(wid * RPT, RPT)], deg_v)
    acc_total = jnp.zeros((16,), jnp.float32)
    for h, zn_hbm in enumerate((zn0_hbm, zn1_hbm)):
        pltpu.sync_copy(zn_hbm.at[pl.ds(wid * RPT, RPT)], zn_v)
        pltpu.sync_copy(az_hbm.at[h, 0, pl.ds(wid * RPT, RPT)], a0_v)
        pltpu.sync_copy(az_hbm.at[h, 1, pl.ds(wid * RPT, RPT)], a1_v)

        def grp_body(g, acc):
            dvec = deg_v[pl.ds(pl.multiple_of(g * 16, 16), 16)]
            for l in range(16):
                dr = dvec[l]
                r = g * 16 + l
                for d in range(D2 // 16):
                    sl = pl.ds(d * 16, 16)
                    t = dr * zn_v[r, sl] - a0_v[r, sl] - a1_v[r, sl]
                    acc = acc + t * t
            return acc

        acc_total = lax.fori_loop(0, RPT // 16, grp_body, acc_total)
    st_v[0, :] = acc_total
    pltpu.sync_copy(st_v.at[0], out_hbm.at[wid])


def kernel(Z, indices, weights, num_nodes):
    pad = NPAD - N
    Zp = jnp.pad(Z, ((0, pad), (0, 0)))
    indp = jnp.pad(indices.astype(jnp.int32), ((0, pad), (0, 0)))
    wp = jnp.pad(weights, ((0, pad), (0, 0)))

    wf = wp.reshape(-1)
    cf = indp.reshape(-1)
    rf = lax.broadcasted_iota(jnp.int32, (NPAD * K,), 0) // K
    lane = lax.broadcasted_iota(jnp.int32, (1, 16), 1)
    w16c = jnp.where((cf % 16)[:, None] == lane, wf[:, None],
                     0.0).reshape(32, EBT, 128, 16)
    idxc = (cf // 16).reshape(32, EBT, 128)
    w16r = jnp.where((rf % 16)[:, None] == lane, wf[:, None],
                     0.0).reshape(32, EBT, 128, 16)
    idxr = (rf // 16).reshape(32, EBT, 128)
    selfidx = jnp.arange(NPAD, dtype=jnp.int32).reshape(32, NCH, 64)
    zeros1 = jnp.zeros((G16, 16), jnp.float32)

    degp = _phase_a1(w16c, idxc, w16r, idxr, zeros1)
    zn0, zn1, deg = _phase_a2(Zp[:, :D2], Zp[:, D2:], degp.reshape(2, NPAD))
    az = _phase_b(zn0, zn1, indp, wp, selfidx)
    partials = _phase_c(zn0, zn1, deg, az)
    smooth = jnp.sum(partials)
    return LAM * smooth / (num_nodes + 1e-08)


# speedup vs baseline: 3.3937x; 1.1986x over previous
import functools

import jax
import jax.numpy as jnp
from jax import lax
from jax.experimental import pallas as pl
from jax.experimental.pallas import tpu as pltpu
from jax.experimental.pallas import tpu_sc as plsc

LAM = 0.001

_mesh = plsc.VectorSubcoreMesh(core_axis_name="c", subcore_axis_name="s")

N = 10000
K = 32
D = 128
D2 = D // 2
NPAD = 10240
G16 = NPAD // 16
RPT = NPAD // 32
EB = (G16 * K) // 128
NCH = RPT // 64


def _newton_rsqrt(x):
    xi = lax.bitcast_convert_type(x, jnp.int32)
    yi = jnp.int32(0x5F3759DF) - lax.shift_right_logical(xi, jnp.int32(1))
    y = lax.bitcast_convert_type(yi, jnp.float32)
    for _ in range(3):
        y = y * (1.5 - 0.5 * x * y * y)
    return y


EBT = (NPAD * K) // (32 * 128)


@functools.partial(
    pl.kernel,
    out_type=jax.ShapeDtypeStruct((2, G16, 16), jnp.float32),
    mesh=_mesh,
    compiler_params=pltpu.CompilerParams(use_tc_tiling_on_sc=False),
    scratch_types=[
        pltpu.VMEM((16, 128, 16), jnp.float32),
        pltpu.VMEM((EBT, 128), jnp.int32),
        pltpu.VMEM_SHARED((G16, 16), jnp.float32),
    ],
)
def _phase_a1(w16c_hbm, idxc_hbm, w16r_hbm, idxr_hbm, zeros_hbm, deg_hbm,
              pay_v, idx_v, deg_sh):
    c = lax.axis_index("c")
    s = lax.axis_index("s")
    wid = s * 2 + c

    pltpu.sync_copy(zeros_hbm.at[pl.ds(s * 40, 40)],
                    deg_sh.at[pl.ds(s * 40, 40)])
    plsc.subcore_barrier()

    for w16_hbm, id_hbm in ((w16c_hbm, idxc_hbm), (w16r_hbm, idxr_hbm)):
        pltpu.sync_copy(id_hbm.at[wid], idx_v)

        def chunk_body(ch, carry):
            pltpu.sync_copy(w16_hbm.at[wid, pl.ds(ch * 16, 16)], pay_v)
            for j in range(16):
                pltpu.sync_copy(pay_v.at[j],
                                deg_sh.at[idx_v.at[ch * 16 + j]], add=True)
            return carry

        lax.fori_loop(0, EBT // 16, chunk_body, jnp.int32(0))
    plsc.subcore_barrier()

    pltpu.sync_copy(deg_sh.at[pl.ds(s * 40, 40)],
                    deg_hbm.at[c, pl.ds(s * 40, 40)])


@functools.partial(
    pl.kernel,
    out_type=(
        jax.ShapeDtypeStruct((NPAD, D2), jnp.float32),
        jax.ShapeDtypeStruct((NPAD, D2), jnp.float32),
        jax.ShapeDtypeStruct((NPAD,), jnp.float32),
    ),
    mesh=_mesh,
    compiler_params=pltpu.CompilerParams(use_tc_tiling_on_sc=False),
    scratch_types=[
        pltpu.VMEM((RPT, D2), jnp.float32),
        pltpu.VMEM((RPT, D2), jnp.float32),
        pltpu.VMEM((RPT,), jnp.float32),
        pltpu.VMEM((RPT,), jnp.float32),
        pltpu.VMEM((RPT,), jnp.float32),
    ],
)
def _phase_a2(z0_hbm, z1_hbm, degp_hbm, zn0_hbm, zn1_hbm, deg_hbm,
              z0_v, z1_v, dp_v, dq_v, dinv_v):
    c = lax.axis_index("c")
    s = lax.axis_index("s")
    wid = s * 2 + c

    pltpu.sync_copy(degp_hbm.at[0, pl.ds(wid * RPT, RPT)], dp_v)
    pltpu.sync_copy(degp_hbm.at[1, pl.ds(wid * RPT, RPT)], dq_v)
    pltpu.sync_copy(z0_hbm.at[pl.ds(wid * RPT, RPT)], z0_v)
    pltpu.sync_copy(z1_hbm.at[pl.ds(wid * RPT, RPT)], z1_v)
    for j in range(RPT // 16):
        sl = pl.ds(j * 16, 16)
        dh = (dp_v[sl] + dq_v[sl]) * 0.5
        dp_v[sl] = dh
        dinv_v[sl] = _newton_rsqrt(dh + 1e-8)

    def grp_body(g, carry):
        dvec = dinv_v[pl.ds(pl.multiple_of(g * 16, 16), 16)]
        for l in range(16):
            dv = dvec[l]
            r = g * 16 + l
            for d in range(D2 // 16):
                sl = pl.ds(d * 16, 16)
                z0_v[r, sl] = dv * z0_v[r, sl]
                z1_v[r, sl] = dv * z1_v[r, sl]
        return carry

    lax.fori_loop(0, RPT // 16, grp_body, jnp.int32(0))
    pltpu.sync_copy(z0_v, zn0_hbm.at[pl.ds(wid * RPT, RPT)])
    pltpu.sync_copy(z1_v, zn1_hbm.at[pl.ds(wid * RPT, RPT)])
    pltpu.sync_copy(dp_v, deg_hbm.at[pl.ds(wid * RPT, RPT)])


@functools.partial(
    pl.kernel,
    out_type=jax.ShapeDtypeStruct((2, 2, NPAD, D2), jnp.float32),
    mesh=_mesh,
    compiler_params=pltpu.CompilerParams(use_tc_tiling_on_sc=False),
    scratch_types=[
        pltpu.VMEM((16, 128), jnp.int32),
        pltpu.VMEM((64, K), jnp.float32),
        pltpu.VMEM((64, D2), jnp.float32),
        pltpu.VMEM((64, D2), jnp.float32),
        pltpu.VMEM((128, D2), jnp.float32),
        pltpu.VMEM((128, D2), jnp.float32),
        pltpu.VMEM((NCH, 64), jnp.int32),
        pltpu.VMEM_SHARED((NPAD, D2), jnp.float32),
        pltpu.SemaphoreType.DMA,
    ],
)
def _phase_b(zn0_hbm, zn1_hbm, indb_hbm, w_hbm, selfidx_hbm, az_hbm,
             ind_v, w_v, zn_v, azo_v, gbuf, sbuf, sidx_v, az_sh, sem):
    c = lax.axis_index("c")
    s = lax.axis_index("s")
    wid = s * 2 + c

    pltpu.sync_copy(selfidx_hbm.at[wid], sidx_v)

    for h, zn_hbm in enumerate((zn0_hbm, zn1_hbm)):
        for k in range(128):
            for d in range(D2 // 16):
                sbuf[k, pl.ds(d * 16, 16)] = jnp.zeros((16,), jnp.float32)
        for j in range(G16 // 128):
            pltpu.sync_copy(sbuf, az_sh.at[pl.ds(s * G16 + j * 128, 128)])
        plsc.subcore_barrier()

        def chunk_body(ch, carry):
            base = wid * RPT + ch * 64
            pltpu.sync_copy(indb_hbm.at[pl.ds(wid * 80 + ch * 16, 16)], ind_v)
            pltpu.sync_copy(w_hbm.at[pl.ds(base, 64)], w_v)
            pltpu.sync_copy(zn_hbm.at[pl.ds(base, 64)], zn_v)

            def grp_body(g, carry2):
                pltpu.async_copy(zn_hbm.at[ind_v.at[g]], gbuf, sem).wait()
                for q in range(4):
                    r = g * 4 + q
                    acc = [jnp.zeros((16,), jnp.float32)
                           for _ in range(D2 // 16)]
                    zrow = [zn_v[r, pl.ds(d * 16, 16)]
                            for d in range(D2 // 16)]
                    wrow = [w_v[r, pl.ds(0, 16)], w_v[r, pl.ds(16, 16)]]
                    for k in range(K):
                        wk = wrow[k // 16][k % 16] * 0.5
                        for d in range(D2 // 16):
                            sl = pl.ds(d * 16, 16)
                            acc[d] = acc[d] + wk * gbuf[q * K + k, sl]
                            sbuf[q * K + k, sl] = wk * zrow[d]
                    for d in range(D2 // 16):
                        azo_v[r, pl.ds(d * 16, 16)] = acc[d]
                pltpu.sync_copy(sbuf, az_sh.at[ind_v.at[g]], add=True)
                return carry2

            lax.fori_loop(0, 16, grp_body, jnp.int32(0))
            pltpu.sync_copy(azo_v, az_sh.at[sidx_v.at[ch]], add=True)
            return carry

        lax.fori_loop(0, NCH, chunk_body, jnp.int32(0))
        plsc.subcore_barrier()

        pltpu.sync_copy(az_sh.at[pl.ds(s * G16, G16)],
                        az_hbm.at[h, c, pl.ds(s * G16, G16)])


@functools.partial(
    pl.kernel,
    out_type=jax.ShapeDtypeStruct((32, 16), jnp.float32),
    mesh=_mesh,
    compiler_params=pltpu.CompilerParams(use_tc_tiling_on_sc=False),
    scratch_types=[
        pltpu.VMEM((RPT, D2), jnp.float32),
        pltpu.VMEM((RPT, D2), jnp.float32),
        pltpu.VMEM((RPT, D2), jnp.float32),
        pltpu.VMEM((RPT,), jnp.float32),
        pltpu.VMEM((1, 16), jnp.float32),
    ],
)
def _phase_c(zn0_hbm, zn1_hbm, deg_hbm, az_hbm, out_hbm,
             zn_v, a0_v, a1_v, deg_v, st_v):
    c = lax.axis_index("c")
    s = lax.axis_index("s")
    wid = s * 2 + c

    pltpu.sync_copy(deg_hbm.at[pl.ds(wid * RPT, RPT)], deg_v)
    acc_total = jnp.zeros((16,), jnp.float32)
    for h, zn_hbm in enumerate((zn0_hbm, zn1_hbm)):
        pltpu.sync_copy(zn_hbm.at[pl.ds(wid * RPT, RPT)], zn_v)
        pltpu.sync_copy(az_hbm.at[h, 0, pl.ds(wid * RPT, RPT)], a0_v)
        pltpu.sync_copy(az_hbm.at[h, 1, pl.ds(wid * RPT, RPT)], a1_v)

        def grp_body(g, acc):
            dvec = deg_v[pl.ds(pl.multiple_of(g * 16, 16), 16)]
            for l in range(16):
                dr = dvec[l]
                r = g * 16 + l
                for d in range(D2 // 16):
                    sl = pl.ds(d * 16, 16)
                    t = dr * zn_v[r, sl] - a0_v[r, sl] - a1_v[r, sl]
                    acc = acc + t * t
            return acc

        acc_total = lax.fori_loop(0, RPT // 16, grp_body, acc_total)
    st_v[0, :] = acc_total
    pltpu.sync_copy(st_v.at[0], out_hbm.at[wid])


def kernel(Z, indices, weights, num_nodes):
    pad = NPAD - N
    Zp = jnp.pad(Z, ((0, pad), (0, 0)))
    indp = jnp.pad(indices.astype(jnp.int32), ((0, pad), (0, 0)))
    wp = jnp.pad(weights, ((0, pad), (0, 0)))

    wf = wp.reshape(-1)
    cf = indp.reshape(-1)
    rf = lax.broadcasted_iota(jnp.int32, (NPAD * K,), 0) // K
    lane = lax.broadcasted_iota(jnp.int32, (1, 16), 1)
    w16c = jnp.where((cf % 16)[:, None] == lane, wf[:, None],
                     0.0).reshape(32, EBT, 128, 16)
    idxc = (cf // 16).reshape(32, EBT, 128)
    w16r = jnp.where((rf % 16)[:, None] == lane, wf[:, None],
                     0.0).reshape(32, EBT, 128, 16)
    idxr = (rf // 16).reshape(32, EBT, 128)
    selfidx = jnp.arange(NPAD, dtype=jnp.int32).reshape(32, NCH, 64)
    zeros1 = jnp.zeros((G16, 16), jnp.float32)

    degp = _phase_a1(w16c, idxc, w16r, idxr, zeros1)
    zn0, zn1, deg = _phase_a2(Zp[:, :D2], Zp[:, D2:], degp.reshape(2, NPAD))
    indb = indp.reshape((NPAD * K) // 128, 128)
    az = _phase_b(zn0, zn1, indb, wp, selfidx)
    partials = _phase_c(zn0, zn1, deg, az)
    smooth = jnp.sum(partials)
    return LAM * smooth / (num_nodes + 1e-08)


# speedup vs baseline: 3.9215x; 1.1555x over previous
import functools

import jax
import jax.numpy as jnp
from jax import lax
from jax.experimental import pallas as pl
from jax.experimental.pallas import tpu as pltpu
from jax.experimental.pallas import tpu_sc as plsc

LAM = 0.001

_mesh = plsc.VectorSubcoreMesh(core_axis_name="c", subcore_axis_name="s")

N = 10000
K = 32
D = 128
D2 = D // 2
NPAD = 10240
G16 = NPAD // 16
RPT = NPAD // 32
EB = (G16 * K) // 128
NCH = RPT // 64


def _newton_rsqrt(x):
    xi = lax.bitcast_convert_type(x, jnp.int32)
    yi = jnp.int32(0x5F3759DF) - lax.shift_right_logical(xi, jnp.int32(1))
    y = lax.bitcast_convert_type(yi, jnp.float32)
    for _ in range(3):
        y = y * (1.5 - 0.5 * x * y * y)
    return y


EBT = (NPAD * K) // (32 * 128)


@functools.partial(
    pl.kernel,
    out_type=jax.ShapeDtypeStruct((2, G16, 16), jnp.float32),
    mesh=_mesh,
    compiler_params=pltpu.CompilerParams(use_tc_tiling_on_sc=False),
    scratch_types=[
        pltpu.VMEM((16, 128, 16), jnp.float32),
        pltpu.VMEM((EBT, 128), jnp.int32),
        pltpu.VMEM_SHARED((G16, 16), jnp.float32),
    ],
)
def _phase_a1(w16c_hbm, idxc_hbm, w16r_hbm, idxr_hbm, zeros_hbm, deg_hbm,
              pay_v, idx_v, deg_sh):
    c = lax.axis_index("c")
    s = lax.axis_index("s")
    wid = s * 2 + c

    pltpu.sync_copy(zeros_hbm.at[pl.ds(s * 40, 40)],
                    deg_sh.at[pl.ds(s * 40, 40)])
    plsc.subcore_barrier()

    for w16_hbm, id_hbm in ((w16c_hbm, idxc_hbm), (w16r_hbm, idxr_hbm)):
        pltpu.sync_copy(id_hbm.at[wid], idx_v)

        def chunk_body(ch, carry):
            pltpu.sync_copy(w16_hbm.at[wid, pl.ds(ch * 16, 16)], pay_v)
            for j in range(16):
                pltpu.sync_copy(pay_v.at[j],
                                deg_sh.at[idx_v.at[ch * 16 + j]], add=True)
            return carry

        lax.fori_loop(0, EBT // 16, chunk_body, jnp.int32(0))
    plsc.subcore_barrier()

    pltpu.sync_copy(deg_sh.at[pl.ds(s * 40, 40)],
                    deg_hbm.at[c, pl.ds(s * 40, 40)])


@functools.partial(
    pl.kernel,
    out_type=(
        jax.ShapeDtypeStruct((NPAD, D2), jnp.float32),
        jax.ShapeDtypeStruct((NPAD, D2), jnp.float32),
        jax.ShapeDtypeStruct((NPAD,), jnp.float32),
    ),
    mesh=_mesh,
    compiler_params=pltpu.CompilerParams(use_tc_tiling_on_sc=False),
    scratch_types=[
        pltpu.VMEM((RPT, D2), jnp.float32),
        pltpu.VMEM((RPT, D2), jnp.float32),
        pltpu.VMEM((RPT,), jnp.float32),
        pltpu.VMEM((RPT,), jnp.float32),
        pltpu.VMEM((RPT,), jnp.float32),
    ],
)
def _phase_a2(z0_hbm, z1_hbm, degp_hbm, zn0_hbm, zn1_hbm, deg_hbm,
              z0_v, z1_v, dp_v, dq_v, dinv_v):
    c = lax.axis_index("c")
    s = lax.axis_index("s")
    wid = s * 2 + c

    pltpu.sync_copy(degp_hbm.at[0, pl.ds(wid * RPT, RPT)], dp_v)
    pltpu.sync_copy(degp_hbm.at[1, pl.ds(wid * RPT, RPT)], dq_v)
    pltpu.sync_copy(z0_hbm.at[pl.ds(wid * RPT, RPT)], z0_v)
    pltpu.sync_copy(z1_hbm.at[pl.ds(wid * RPT, RPT)], z1_v)
    for j in range(RPT // 16):
        sl = pl.ds(j * 16, 16)
        dh = (dp_v[sl] + dq_v[sl]) * 0.5
        dp_v[sl] = dh
        dinv_v[sl] = _newton_rsqrt(dh + 1e-8)

    def grp_body(g, carry):
        dvec = dinv_v[pl.ds(pl.multiple_of(g * 16, 16), 16)]
        for l in range(16):
            dv = dvec[l]
            r = g * 16 + l
            for d in range(D2 // 16):
                sl = pl.ds(d * 16, 16)
                z0_v[r, sl] = dv * z0_v[r, sl]
                z1_v[r, sl] = dv * z1_v[r, sl]
        return carry

    lax.fori_loop(0, RPT // 16, grp_body, jnp.int32(0))
    pltpu.sync_copy(z0_v, zn0_hbm.at[pl.ds(wid * RPT, RPT)])
    pltpu.sync_copy(z1_v, zn1_hbm.at[pl.ds(wid * RPT, RPT)])
    pltpu.sync_copy(dp_v, deg_hbm.at[pl.ds(wid * RPT, RPT)])


@functools.partial(
    pl.kernel,
    out_type=jax.ShapeDtypeStruct((2, 2, NPAD, D2), jnp.float32),
    mesh=_mesh,
    compiler_params=pltpu.CompilerParams(use_tc_tiling_on_sc=False),
    scratch_types=[
        pltpu.VMEM((16, 128), jnp.int32),
        pltpu.VMEM((64, K), jnp.float32),
        pltpu.VMEM((64, D2), jnp.float32),
        pltpu.VMEM((64, D2), jnp.float32),
        pltpu.VMEM((2, 128, D2), jnp.float32),
        pltpu.VMEM((2, 128, D2), jnp.float32),
        pltpu.VMEM((NCH, 64), jnp.int32),
        pltpu.VMEM_SHARED((NPAD, D2), jnp.float32),
        pltpu.SemaphoreType.DMA,
        pltpu.SemaphoreType.DMA,
    ],
)
def _phase_b(zn0_hbm, zn1_hbm, indb_hbm, w_hbm, selfidx_hbm, az_hbm,
             ind_v, w_v, zn_v, azo_v, gbuf, sbuf, sidx_v, az_sh, sem, sem_s):
    c = lax.axis_index("c")
    s = lax.axis_index("s")
    wid = s * 2 + c

    pltpu.sync_copy(selfidx_hbm.at[wid], sidx_v)

    for h, zn_hbm in enumerate((zn0_hbm, zn1_hbm)):
        for k in range(128):
            for d in range(D2 // 16):
                sbuf[0, k, pl.ds(d * 16, 16)] = jnp.zeros((16,), jnp.float32)
        for j in range(G16 // 128):
            pltpu.sync_copy(sbuf.at[0], az_sh.at[pl.ds(s * G16 + j * 128, 128)])
        plsc.subcore_barrier()

        def chunk_body(ch, carry):
            base = wid * RPT + ch * 64
            pltpu.sync_copy(indb_hbm.at[pl.ds(wid * 80 + ch * 16, 16)], ind_v)
            pltpu.sync_copy(w_hbm.at[pl.ds(base, 64)], w_v)
            pltpu.sync_copy(zn_hbm.at[pl.ds(base, 64)], zn_v)
            pltpu.async_copy(zn_hbm.at[ind_v.at[0]], gbuf.at[0], sem)

            def grp_body(g, carry2):
                @pl.when(g < 15)
                def _():
                    pltpu.async_copy(zn_hbm.at[ind_v.at[g + 1]],
                                     gbuf.at[(g + 1) % 2], sem)
                gcur = gbuf.at[g % 2]
                scur = sbuf.at[g % 2]
                pltpu.make_async_copy(zn_hbm.at[ind_v.at[g]], gcur, sem).wait()
                @pl.when(g >= 2)
                def _():
                    pltpu.make_async_copy(scur, az_sh.at[ind_v.at[g]],
                                          sem_s).wait()
                for q in range(4):
                    r = g * 4 + q
                    acc = [jnp.zeros((16,), jnp.float32)
                           for _ in range(D2 // 16)]
                    zrow = [zn_v[r, pl.ds(d * 16, 16)]
                            for d in range(D2 // 16)]
                    wrow = [w_v[r, pl.ds(0, 16)], w_v[r, pl.ds(16, 16)]]
                    for k in range(K):
                        wk = wrow[k // 16][k % 16] * 0.5
                        for d in range(D2 // 16):
                            sl = pl.ds(d * 16, 16)
                            acc[d] = acc[d] + wk * gcur[q * K + k, sl]
                            scur[q * K + k, sl] = wk * zrow[d]
                    for d in range(D2 // 16):
                        azo_v[r, pl.ds(d * 16, 16)] = acc[d]
                pltpu.async_copy(scur, az_sh.at[ind_v.at[g]], sem_s, add=True)
                return carry2

            lax.fori_loop(0, 16, grp_body, jnp.int32(0))
            for t in range(2):
                pltpu.make_async_copy(sbuf.at[t], az_sh.at[ind_v.at[14 + t]],
                                      sem_s).wait()
            pltpu.sync_copy(azo_v, az_sh.at[sidx_v.at[ch]], add=True)
            return carry

        lax.fori_loop(0, NCH, chunk_body, jnp.int32(0))
        plsc.subcore_barrier()

        pltpu.sync_copy(az_sh.at[pl.ds(s * G16, G16)],
                        az_hbm.at[h, c, pl.ds(s * G16, G16)])


@functools.partial(
    pl.kernel,
    out_type=jax.ShapeDtypeStruct((32, 16), jnp.float32),
    mesh=_mesh,
    compiler_params=pltpu.CompilerParams(use_tc_tiling_on_sc=False),
    scratch_types=[
        pltpu.VMEM((RPT, D2), jnp.float32),
        pltpu.VMEM((RPT, D2), jnp.float32),
        pltpu.VMEM((RPT, D2), jnp.float32),
        pltpu.VMEM((RPT,), jnp.float32),
        pltpu.VMEM((1, 16), jnp.float32),
    ],
)
def _phase_c(zn0_hbm, zn1_hbm, deg_hbm, az_hbm, out_hbm,
             zn_v, a0_v, a1_v, deg_v, st_v):
    c = lax.axis_index("c")
    s = lax.axis_index("s")
    wid = s * 2 + c

    pltpu.sync_copy(deg_hbm.at[pl.ds(wid * RPT, RPT)], deg_v)
    acc_total = jnp.zeros((16,), jnp.float32)
    for h, zn_hbm in enumerate((zn0_hbm, zn1_hbm)):
        pltpu.sync_copy(zn_hbm.at[pl.ds(wid * RPT, RPT)], zn_v)
        pltpu.sync_copy(az_hbm.at[h, 0, pl.ds(wid * RPT, RPT)], a0_v)
        pltpu.sync_copy(az_hbm.at[h, 1, pl.ds(wid * RPT, RPT)], a1_v)

        def grp_body(g, acc):
            dvec = deg_v[pl.ds(pl.multiple_of(g * 16, 16), 16)]
            for l in range(16):
                dr = dvec[l]
                r = g * 16 + l
                for d in range(D2 // 16):
                    sl = pl.ds(d * 16, 16)
                    t = dr * zn_v[r, sl] - a0_v[r, sl] - a1_v[r, sl]
                    acc = acc + t * t
            return acc

        acc_total = lax.fori_loop(0, RPT // 16, grp_body, acc_total)
    st_v[0, :] = acc_total
    pltpu.sync_copy(st_v.at[0], out_hbm.at[wid])


def kernel(Z, indices, weights, num_nodes):
    pad = NPAD - N
    Zp = jnp.pad(Z, ((0, pad), (0, 0)))
    indp = jnp.pad(indices.astype(jnp.int32), ((0, pad), (0, 0)))
    wp = jnp.pad(weights, ((0, pad), (0, 0)))

    wf = wp.reshape(-1)
    cf = indp.reshape(-1)
    rf = lax.broadcasted_iota(jnp.int32, (NPAD * K,), 0) // K
    lane = lax.broadcasted_iota(jnp.int32, (1, 16), 1)
    w16c = jnp.where((cf % 16)[:, None] == lane, wf[:, None],
                     0.0).reshape(32, EBT, 128, 16)
    idxc = (cf // 16).reshape(32, EBT, 128)
    w16r = jnp.where((rf % 16)[:, None] == lane, wf[:, None],
                     0.0).reshape(32, EBT, 128, 16)
    idxr = (rf // 16).reshape(32, EBT, 128)
    selfidx = jnp.arange(NPAD, dtype=jnp.int32).reshape(32, NCH, 64)
    zeros1 = jnp.zeros((G16, 16), jnp.float32)

    degp = _phase_a1(w16c, idxc, w16r, idxr, zeros1)
    zn0, zn1, deg = _phase_a2(Zp[:, :D2], Zp[:, D2:], degp.reshape(2, NPAD))
    indb = indp.reshape((NPAD * K) // 128, 128)
    az = _phase_b(zn0, zn1, indb, wp, selfidx)
    partials = _phase_c(zn0, zn1, deg, az)
    smooth = jnp.sum(partials)
    return LAM * smooth / (num_nodes + 1e-08)


# speedup vs baseline: 3.9218x; 1.0001x over previous
import functools

import jax
import jax.numpy as jnp
from jax import lax
from jax.experimental import pallas as pl
from jax.experimental.pallas import tpu as pltpu
from jax.experimental.pallas import tpu_sc as plsc

LAM = 0.001

_mesh = plsc.VectorSubcoreMesh(core_axis_name="c", subcore_axis_name="s")

N = 10000
K = 32
D = 128
D2 = D // 2
NPAD = 10240
G16 = NPAD // 16
RPT = NPAD // 32
EB = (G16 * K) // 128
NCH = RPT // 64


def _newton_rsqrt(x):
    xi = lax.bitcast_convert_type(x, jnp.int32)
    yi = jnp.int32(0x5F3759DF) - lax.shift_right_logical(xi, jnp.int32(1))
    y = lax.bitcast_convert_type(yi, jnp.float32)
    for _ in range(3):
        y = y * (1.5 - 0.5 * x * y * y)
    return y


EBT = (NPAD * K) // (32 * 128)


@functools.partial(
    pl.kernel,
    out_type=jax.ShapeDtypeStruct((2, G16, 16), jnp.float32),
    mesh=_mesh,
    compiler_params=pltpu.CompilerParams(use_tc_tiling_on_sc=False),
    scratch_types=[
        pltpu.VMEM((16, 128, 16), jnp.float32),
        pltpu.VMEM((EBT, 128), jnp.int32),
        pltpu.VMEM_SHARED((G16, 16), jnp.float32),
    ],
)
def _phase_a1(w16c_hbm, idxc_hbm, w16r_hbm, idxr_hbm, zeros_hbm, deg_hbm,
              pay_v, idx_v, deg_sh):
    c = lax.axis_index("c")
    s = lax.axis_index("s")
    wid = s * 2 + c

    pltpu.sync_copy(zeros_hbm.at[pl.ds(s * 40, 40)],
                    deg_sh.at[pl.ds(s * 40, 40)])
    plsc.subcore_barrier()

    for w16_hbm, id_hbm in ((w16c_hbm, idxc_hbm), (w16r_hbm, idxr_hbm)):
        pltpu.sync_copy(id_hbm.at[wid], idx_v)

        def chunk_body(ch, carry):
            pltpu.sync_copy(w16_hbm.at[wid, pl.ds(ch * 16, 16)], pay_v)
            for j in range(16):
                pltpu.sync_copy(pay_v.at[j],
                                deg_sh.at[idx_v.at[ch * 16 + j]], add=True)
            return carry

        lax.fori_loop(0, EBT // 16, chunk_body, jnp.int32(0))
    plsc.subcore_barrier()

    pltpu.sync_copy(deg_sh.at[pl.ds(s * 40, 40)],
                    deg_hbm.at[c, pl.ds(s * 40, 40)])


@functools.partial(
    pl.kernel,
    out_type=(
        jax.ShapeDtypeStruct((NPAD, D2), jnp.float32),
        jax.ShapeDtypeStruct((NPAD, D2), jnp.float32),
        jax.ShapeDtypeStruct((NPAD,), jnp.float32),
    ),
    mesh=_mesh,
    compiler_params=pltpu.CompilerParams(use_tc_tiling_on_sc=False),
    scratch_types=[
        pltpu.VMEM((RPT, D2), jnp.float32),
        pltpu.VMEM((RPT, D2), jnp.float32),
        pltpu.VMEM((RPT,), jnp.float32),
        pltpu.VMEM((RPT,), jnp.float32),
        pltpu.VMEM((RPT,), jnp.float32),
    ],
)
def _phase_a2(z0_hbm, z1_hbm, degp_hbm, zn0_hbm, zn1_hbm, deg_hbm,
              z0_v, z1_v, dp_v, dq_v, dinv_v):
    c = lax.axis_index("c")
    s = lax.axis_index("s")
    wid = s * 2 + c

    pltpu.sync_copy(degp_hbm.at[0, pl.ds(wid * RPT, RPT)], dp_v)
    pltpu.sync_copy(degp_hbm.at[1, pl.ds(wid * RPT, RPT)], dq_v)
    pltpu.sync_copy(z0_hbm.at[pl.ds(wid * RPT, RPT)], z0_v)
    pltpu.sync_copy(z1_hbm.at[pl.ds(wid * RPT, RPT)], z1_v)
    for j in range(RPT // 16):
        sl = pl.ds(j * 16, 16)
        dh = (dp_v[sl] + dq_v[sl]) * 0.5
        dp_v[sl] = dh
        dinv_v[sl] = _newton_rsqrt(dh + 1e-8)

    def grp_body(g, carry):
        dvec = dinv_v[pl.ds(pl.multiple_of(g * 16, 16), 16)]
        for l in range(16):
            dv = dvec[l]
            r = g * 16 + l
            for d in range(D2 // 16):
                sl = pl.ds(d * 16, 16)
                z0_v[r, sl] = dv * z0_v[r, sl]
                z1_v[r, sl] = dv * z1_v[r, sl]
        return carry

    lax.fori_loop(0, RPT // 16, grp_body, jnp.int32(0))
    pltpu.sync_copy(z0_v, zn0_hbm.at[pl.ds(wid * RPT, RPT)])
    pltpu.sync_copy(z1_v, zn1_hbm.at[pl.ds(wid * RPT, RPT)])
    pltpu.sync_copy(dp_v, deg_hbm.at[pl.ds(wid * RPT, RPT)])


@functools.partial(
    pl.kernel,
    out_type=jax.ShapeDtypeStruct((2, 2, NPAD, D2), jnp.float32),
    mesh=_mesh,
    compiler_params=pltpu.CompilerParams(use_tc_tiling_on_sc=False),
    scratch_types=[
        pltpu.VMEM((16, 128), jnp.int32),
        pltpu.VMEM((64, K), jnp.float32),
        pltpu.VMEM((64, D2), jnp.float32),
        pltpu.VMEM((64, D2), jnp.float32),
        pltpu.VMEM((4, 128, D2), jnp.float32),
        pltpu.VMEM((4, 128, D2), jnp.float32),
        pltpu.VMEM((NCH, 64), jnp.int32),
        pltpu.VMEM_SHARED((NPAD, D2), jnp.float32),
        pltpu.SemaphoreType.DMA,
        pltpu.SemaphoreType.DMA,
    ],
)
def _phase_b(zn0_hbm, zn1_hbm, indb_hbm, w_hbm, selfidx_hbm, az_hbm,
             ind_v, w_v, zn_v, azo_v, gbuf, sbuf, sidx_v, az_sh, sem, sem_s):
    c = lax.axis_index("c")
    s = lax.axis_index("s")
    wid = s * 2 + c

    pltpu.sync_copy(selfidx_hbm.at[wid], sidx_v)

    for h, zn_hbm in enumerate((zn0_hbm, zn1_hbm)):
        for k in range(128):
            for d in range(D2 // 16):
                sbuf[0, k, pl.ds(d * 16, 16)] = jnp.zeros((16,), jnp.float32)
        for j in range(G16 // 128):
            pltpu.sync_copy(sbuf.at[0], az_sh.at[pl.ds(s * G16 + j * 128, 128)])
        plsc.subcore_barrier()

        def chunk_body(ch, carry):
            base = wid * RPT + ch * 64
            pltpu.sync_copy(indb_hbm.at[pl.ds(wid * 80 + ch * 16, 16)], ind_v)
            pltpu.sync_copy(w_hbm.at[pl.ds(base, 64)], w_v)
            pltpu.sync_copy(zn_hbm.at[pl.ds(base, 64)], zn_v)
            for t in range(3):
                pltpu.async_copy(zn_hbm.at[ind_v.at[t]], gbuf.at[t], sem)

            def grp_body(g, carry2):
                @pl.when(g < 13)
                def _():
                    pltpu.async_copy(zn_hbm.at[ind_v.at[g + 3]],
                                     gbuf.at[(g + 3) % 4], sem)
                gcur = gbuf.at[g % 4]
                scur = sbuf.at[g % 4]
                pltpu.make_async_copy(zn_hbm.at[ind_v.at[g]], gcur, sem).wait()
                @pl.when(g >= 4)
                def _():
                    pltpu.make_async_copy(scur, az_sh.at[ind_v.at[g]],
                                          sem_s).wait()
                for q in range(4):
                    r = g * 4 + q
                    acc = [jnp.zeros((16,), jnp.float32)
                           for _ in range(D2 // 16)]
                    zrow = [zn_v[r, pl.ds(d * 16, 16)]
                            for d in range(D2 // 16)]
                    wrow = [w_v[r, pl.ds(0, 16)], w_v[r, pl.ds(16, 16)]]
                    for k in range(K):
                        wk = wrow[k // 16][k % 16] * 0.5
                        for d in range(D2 // 16):
                            sl = pl.ds(d * 16, 16)
                            acc[d] = acc[d] + wk * gcur[q * K + k, sl]
                            scur[q * K + k, sl] = wk * zrow[d]
                    for d in range(D2 // 16):
                        azo_v[r, pl.ds(d * 16, 16)] = acc[d]
                pltpu.async_copy(scur, az_sh.at[ind_v.at[g]], sem_s, add=True)
                return carry2

            lax.fori_loop(0, 16, grp_body, jnp.int32(0))
            for t in range(4):
                pltpu.make_async_copy(sbuf.at[t], az_sh.at[ind_v.at[12 + t]],
                                      sem_s).wait()
            pltpu.sync_copy(azo_v, az_sh.at[sidx_v.at[ch]], add=True)
            return carry

        lax.fori_loop(0, NCH, chunk_body, jnp.int32(0))
        plsc.subcore_barrier()

        pltpu.sync_copy(az_sh.at[pl.ds(s * G16, G16)],
                        az_hbm.at[h, c, pl.ds(s * G16, G16)])


@functools.partial(
    pl.kernel,
    out_type=jax.ShapeDtypeStruct((32, 16), jnp.float32),
    mesh=_mesh,
    compiler_params=pltpu.CompilerParams(use_tc_tiling_on_sc=False),
    scratch_types=[
        pltpu.VMEM((RPT, D2), jnp.float32),
        pltpu.VMEM((RPT, D2), jnp.float32),
        pltpu.VMEM((RPT, D2), jnp.float32),
        pltpu.VMEM((RPT,), jnp.float32),
        pltpu.VMEM((1, 16), jnp.float32),
    ],
)
def _phase_c(zn0_hbm, zn1_hbm, deg_hbm, az_hbm, out_hbm,
             zn_v, a0_v, a1_v, deg_v, st_v):
    c = lax.axis_index("c")
    s = lax.axis_index("s")
    wid = s * 2 + c

    pltpu.sync_copy(deg_hbm.at[pl.ds(wid * RPT, RPT)], deg_v)
    acc_total = jnp.zeros((16,), jnp.float32)
    for h, zn_hbm in enumerate((zn0_hbm, zn1_hbm)):
        pltpu.sync_copy(zn_hbm.at[pl.ds(wid * RPT, RPT)], zn_v)
        pltpu.sync_copy(az_hbm.at[h, 0, pl.ds(wid * RPT, RPT)], a0_v)
        pltpu.sync_copy(az_hbm.at[h, 1, pl.ds(wid * RPT, RPT)], a1_v)

        def grp_body(g, acc):
            dvec = deg_v[pl.ds(pl.multiple_of(g * 16, 16), 16)]
            for l in range(16):
                dr = dvec[l]
                r = g * 16 + l
                for d in range(D2 // 16):
                    sl = pl.ds(d * 16, 16)
                    t = dr * zn_v[r, sl] - a0_v[r, sl] - a1_v[r, sl]
                    acc = acc + t * t
            return acc

        acc_total = lax.fori_loop(0, RPT // 16, grp_body, acc_total)
    st_v[0, :] = acc_total
    pltpu.sync_copy(st_v.at[0], out_hbm.at[wid])


def kernel(Z, indices, weights, num_nodes):
    pad = NPAD - N
    Zp = jnp.pad(Z, ((0, pad), (0, 0)))
    indp = jnp.pad(indices.astype(jnp.int32), ((0, pad), (0, 0)))
    wp = jnp.pad(weights, ((0, pad), (0, 0)))

    wf = wp.reshape(-1)
    cf = indp.reshape(-1)
    rf = lax.broadcasted_iota(jnp.int32, (NPAD * K,), 0) // K
    lane = lax.broadcasted_iota(jnp.int32, (1, 16), 1)
    w16c = jnp.where((cf % 16)[:, None] == lane, wf[:, None],
                     0.0).reshape(32, EBT, 128, 16)
    idxc = (cf // 16).reshape(32, EBT, 128)
    w16r = jnp.where((rf % 16)[:, None] == lane, wf[:, None],
                     0.0).reshape(32, EBT, 128, 16)
    idxr = (rf // 16).reshape(32, EBT, 128)
    selfidx = jnp.arange(NPAD, dtype=jnp.int32).reshape(32, NCH, 64)
    zeros1 = jnp.zeros((G16, 16), jnp.float32)

    degp = _phase_a1(w16c, idxc, w16r, idxr, zeros1)
    zn0, zn1, deg = _phase_a2(Zp[:, :D2], Zp[:, D2:], degp.reshape(2, NPAD))
    indb = indp.reshape((NPAD * K) // 128, 128)
    az = _phase_b(zn0, zn1, indb, wp, selfidx)
    partials = _phase_c(zn0, zn1, deg, az)
    smooth = jnp.sum(partials)
    return LAM * smooth / (num_nodes + 1e-08)


# speedup vs baseline: 4.7486x; 1.2108x over previous
import functools

import jax
import jax.numpy as jnp
from jax import lax
from jax.experimental import pallas as pl
from jax.experimental.pallas import tpu as pltpu
from jax.experimental.pallas import tpu_sc as plsc

LAM = 0.001

_mesh = plsc.VectorSubcoreMesh(core_axis_name="c", subcore_axis_name="s")

N = 10000
K = 32
D = 128
D2 = D // 2
NPAD = 10240
G16 = NPAD // 16
RPT = NPAD // 32
EB = (G16 * K) // 128
NCH = RPT // 64


def _newton_rsqrt(x):
    xi = lax.bitcast_convert_type(x, jnp.int32)
    yi = jnp.int32(0x5F3759DF) - lax.shift_right_logical(xi, jnp.int32(1))
    y = lax.bitcast_convert_type(yi, jnp.float32)
    for _ in range(3):
        y = y * (1.5 - 0.5 * x * y * y)
    return y


EBT = (NPAD * K) // (32 * 128)


@functools.partial(
    pl.kernel,
    out_type=jax.ShapeDtypeStruct((2, G16, 16), jnp.float32),
    mesh=_mesh,
    compiler_params=pltpu.CompilerParams(use_tc_tiling_on_sc=False),
    scratch_types=[
        pltpu.VMEM((16, 128, 16), jnp.float32),
        pltpu.VMEM((EBT, 128), jnp.int32),
        pltpu.VMEM((20, K, 16), jnp.float32),
        pltpu.VMEM((20, 16), jnp.float32),
        pltpu.VMEM((20,), jnp.int32),
        pltpu.VMEM_SHARED((G16, 16), jnp.float32),
    ],
)
def _phase_a1(w16c_hbm, idxc_hbm, wt_hbm, gidx_hbm, zeros_hbm, deg_hbm,
              pay_v, idx_v, wt_v, racc_v, gidx_v, deg_sh):
    c = lax.axis_index("c")
    s = lax.axis_index("s")
    wid = s * 2 + c

    pltpu.sync_copy(zeros_hbm.at[pl.ds(s * 40, 40)],
                    deg_sh.at[pl.ds(s * 40, 40)])

    pltpu.sync_copy(wt_hbm.at[pl.ds(wid * 20, 20)], wt_v)
    pltpu.sync_copy(gidx_hbm.at[wid], gidx_v)
    plsc.subcore_barrier()
    for g in range(20):
        acc = wt_v[g, 0, :]
        for k in range(1, K):
            acc = acc + wt_v[g, k, :]
        racc_v[g, :] = acc
    pltpu.sync_copy(racc_v, deg_sh.at[gidx_v], add=True)

    pltpu.sync_copy(idxc_hbm.at[wid], idx_v)

    def chunk_body(ch, carry):
        pltpu.sync_copy(w16c_hbm.at[wid, pl.ds(ch * 16, 16)], pay_v)
        for j in range(16):
            pltpu.sync_copy(pay_v.at[j],
                            deg_sh.at[idx_v.at[ch * 16 + j]], add=True)
        return carry

    lax.fori_loop(0, EBT // 16, chunk_body, jnp.int32(0))
    plsc.subcore_barrier()

    pltpu.sync_copy(deg_sh.at[pl.ds(s * 40, 40)],
                    deg_hbm.at[c, pl.ds(s * 40, 40)])


@functools.partial(
    pl.kernel,
    out_type=(
        jax.ShapeDtypeStruct((NPAD, D2), jnp.float32),
        jax.ShapeDtypeStruct((NPAD, D2), jnp.float32),
        jax.ShapeDtypeStruct((NPAD,), jnp.float32),
    ),
    mesh=_mesh,
    compiler_params=pltpu.CompilerParams(use_tc_tiling_on_sc=False),
    scratch_types=[
        pltpu.VMEM((RPT, D2), jnp.float32),
        pltpu.VMEM((RPT, D2), jnp.float32),
        pltpu.VMEM((RPT,), jnp.float32),
        pltpu.VMEM((RPT,), jnp.float32),
        pltpu.VMEM((RPT,), jnp.float32),
    ],
)
def _phase_a2(z0_hbm, z1_hbm, degp_hbm, zn0_hbm, zn1_hbm, deg_hbm,
              z0_v, z1_v, dp_v, dq_v, dinv_v):
    c = lax.axis_index("c")
    s = lax.axis_index("s")
    wid = s * 2 + c

    pltpu.sync_copy(degp_hbm.at[0, pl.ds(wid * RPT, RPT)], dp_v)
    pltpu.sync_copy(degp_hbm.at[1, pl.ds(wid * RPT, RPT)], dq_v)
    pltpu.sync_copy(z0_hbm.at[pl.ds(wid * RPT, RPT)], z0_v)
    pltpu.sync_copy(z1_hbm.at[pl.ds(wid * RPT, RPT)], z1_v)
    for j in range(RPT // 16):
        sl = pl.ds(j * 16, 16)
        dh = (dp_v[sl] + dq_v[sl]) * 0.5
        dp_v[sl] = dh
        dinv_v[sl] = _newton_rsqrt(dh + 1e-8)

    def grp_body(g, carry):
        dvec = dinv_v[pl.ds(pl.multiple_of(g * 16, 16), 16)]
        for l in range(16):
            dv = dvec[l]
            r = g * 16 + l
            for d in range(D2 // 16):
                sl = pl.ds(d * 16, 16)
                z0_v[r, sl] = dv * z0_v[r, sl]
                z1_v[r, sl] = dv * z1_v[r, sl]
        return carry

    lax.fori_loop(0, RPT // 16, grp_body, jnp.int32(0))
    pltpu.sync_copy(z0_v, zn0_hbm.at[pl.ds(wid * RPT, RPT)])
    pltpu.sync_copy(z1_v, zn1_hbm.at[pl.ds(wid * RPT, RPT)])
    pltpu.sync_copy(dp_v, deg_hbm.at[pl.ds(wid * RPT, RPT)])


@functools.partial(
    pl.kernel,
    out_type=jax.ShapeDtypeStruct((2, 2, NPAD, D2), jnp.float32),
    mesh=_mesh,
    compiler_params=pltpu.CompilerParams(use_tc_tiling_on_sc=False),
    scratch_types=[
        pltpu.VMEM((16, 128), jnp.int32),
        pltpu.VMEM((64, K), jnp.float32),
        pltpu.VMEM((64, D2), jnp.float32),
        pltpu.VMEM((64, D2), jnp.float32),
        pltpu.VMEM((4, 128, D2), jnp.float32),
        pltpu.VMEM((4, 128, D2), jnp.float32),
        pltpu.VMEM((NCH, 64), jnp.int32),
        pltpu.VMEM_SHARED((NPAD, D2), jnp.float32),
        pltpu.SemaphoreType.DMA,
        pltpu.SemaphoreType.DMA,
    ],
)
def _phase_b(zn0_hbm, zn1_hbm, indb_hbm, w_hbm, selfidx_hbm, az_hbm,
             ind_v, w_v, zn_v, azo_v, gbuf, sbuf, sidx_v, az_sh, sem, sem_s):
    c = lax.axis_index("c")
    s = lax.axis_index("s")
    wid = s * 2 + c

    pltpu.sync_copy(selfidx_hbm.at[wid], sidx_v)

    for h, zn_hbm in enumerate((zn0_hbm, zn1_hbm)):
        for k in range(128):
            for d in range(D2 // 16):
                sbuf[0, k, pl.ds(d * 16, 16)] = jnp.zeros((16,), jnp.float32)
        for j in range(G16 // 128):
            pltpu.sync_copy(sbuf.at[0], az_sh.at[pl.ds(s * G16 + j * 128, 128)])
        plsc.subcore_barrier()

        def chunk_body(ch, carry):
            base = wid * RPT + ch * 64
            pltpu.sync_copy(indb_hbm.at[pl.ds(wid * 80 + ch * 16, 16)], ind_v)
            pltpu.sync_copy(w_hbm.at[pl.ds(base, 64)], w_v)
            pltpu.sync_copy(zn_hbm.at[pl.ds(base, 64)], zn_v)
            for t in range(3):
                pltpu.async_copy(zn_hbm.at[ind_v.at[t]], gbuf.at[t], sem)

            def grp_body(g, carry2):
                @pl.when(g < 13)
                def _():
                    pltpu.async_copy(zn_hbm.at[ind_v.at[g + 3]],
                                     gbuf.at[(g + 3) % 4], sem)
                gcur = gbuf.at[g % 4]
                scur = sbuf.at[g % 4]
                pltpu.make_async_copy(zn_hbm.at[ind_v.at[g]], gcur, sem).wait()
                @pl.when(g >= 4)
                def _():
                    pltpu.make_async_copy(scur, az_sh.at[ind_v.at[g]],
                                          sem_s).wait()
                for q in range(4):
                    r = g * 4 + q
                    acc = [jnp.zeros((16,), jnp.float32)
                           for _ in range(D2 // 16)]
                    zrow = [zn_v[r, pl.ds(d * 16, 16)]
                            for d in range(D2 // 16)]
                    wrow = [w_v[r, pl.ds(0, 16)], w_v[r, pl.ds(16, 16)]]
                    for k in range(K):
                        wk = wrow[k // 16][k % 16] * 0.5
                        for d in range(D2 // 16):
                            sl = pl.ds(d * 16, 16)
                            acc[d] = acc[d] + wk * gcur[q * K + k, sl]
                            scur[q * K + k, sl] = wk * zrow[d]
                    for d in range(D2 // 16):
                        azo_v[r, pl.ds(d * 16, 16)] = acc[d]
                pltpu.async_copy(scur, az_sh.at[ind_v.at[g]], sem_s, add=True)
                return carry2

            lax.fori_loop(0, 16, grp_body, jnp.int32(0))
            for t in range(4):
                pltpu.make_async_copy(sbuf.at[t], az_sh.at[ind_v.at[12 + t]],
                                      sem_s).wait()
            pltpu.sync_copy(azo_v, az_sh.at[sidx_v.at[ch]], add=True)
            return carry

        lax.fori_loop(0, NCH, chunk_body, jnp.int32(0))
        plsc.subcore_barrier()

        pltpu.sync_copy(az_sh.at[pl.ds(s * G16, G16)],
                        az_hbm.at[h, c, pl.ds(s * G16, G16)])


@functools.partial(
    pl.kernel,
    out_type=jax.ShapeDtypeStruct((32, 16), jnp.float32),
    mesh=_mesh,
    compiler_params=pltpu.CompilerParams(use_tc_tiling_on_sc=False),
    scratch_types=[
        pltpu.VMEM((RPT, D2), jnp.float32),
        pltpu.VMEM((RPT, D2), jnp.float32),
        pltpu.VMEM((RPT, D2), jnp.float32),
        pltpu.VMEM((RPT,), jnp.float32),
        pltpu.VMEM((1, 16), jnp.float32),
    ],
)
def _phase_c(zn0_hbm, zn1_hbm, deg_hbm, az_hbm, out_hbm,
             zn_v, a0_v, a1_v, deg_v, st_v):
    c = lax.axis_index("c")
    s = lax.axis_index("s")
    wid = s * 2 + c

    pltpu.sync_copy(deg_hbm.at[pl.ds(wid * RPT, RPT)], deg_v)
    acc_total = jnp.zeros((16,), jnp.float32)
    for h, zn_hbm in enumerate((zn0_hbm, zn1_hbm)):
        pltpu.sync_copy(zn_hbm.at[pl.ds(wid * RPT, RPT)], zn_v)
        pltpu.sync_copy(az_hbm.at[h, 0, pl.ds(wid * RPT, RPT)], a0_v)
        pltpu.sync_copy(az_hbm.at[h, 1, pl.ds(wid * RPT, RPT)], a1_v)

        def grp_body(g, acc):
            dvec = deg_v[pl.ds(pl.multiple_of(g * 16, 16), 16)]
            for l in range(16):
                dr = dvec[l]
                r = g * 16 + l
                for d in range(D2 // 16):
                    sl = pl.ds(d * 16, 16)
                    t = dr * zn_v[r, sl] - a0_v[r, sl] - a1_v[r, sl]
                    acc = acc + t * t
            return acc

        acc_total = lax.fori_loop(0, RPT // 16, grp_body, acc_total)
    st_v[0, :] = acc_total
    pltpu.sync_copy(st_v.at[0], out_hbm.at[wid])


def kernel(Z, indices, weights, num_nodes):
    pad = NPAD - N
    Zp = jnp.pad(Z, ((0, pad), (0, 0)))
    indp = jnp.pad(indices.astype(jnp.int32), ((0, pad), (0, 0)))
    wp = jnp.pad(weights, ((0, pad), (0, 0)))

    wf = wp.reshape(-1)
    cf = indp.reshape(-1)
    lane = lax.broadcasted_iota(jnp.int32, (1, 16), 1)
    w16c = jnp.where((cf % 16)[:, None] == lane, wf[:, None],
                     0.0).reshape(32, EBT, 128, 16)
    idxc = (cf // 16).reshape(32, EBT, 128)
    wtb = wp.reshape(G16, 16, K).transpose(0, 2, 1)
    gidx = jnp.arange(G16, dtype=jnp.int32).reshape(32, 20)
    selfidx = jnp.arange(NPAD, dtype=jnp.int32).reshape(32, NCH, 64)
    zeros1 = jnp.zeros((G16, 16), jnp.float32)

    degp = _phase_a1(w16c, idxc, wtb, gidx, zeros1)
    zn0, zn1, deg = _phase_a2(Zp[:, :D2], Zp[:, D2:], degp.reshape(2, NPAD))
    indb = indp.reshape((NPAD * K) // 128, 128)
    az = _phase_b(zn0, zn1, indb, wp, selfidx)
    partials = _phase_c(zn0, zn1, deg, az)
    smooth = jnp.sum(partials)
    return LAM * smooth / (num_nodes + 1e-08)
